# Initial kernel scaffold; baseline (speedup 1.0000x reference)
import jax
import jax.numpy as jnp
from jax.experimental import pallas as pl


def kernel(pos, emb_main, emb_tag, emb_period, emb_group, W_lin, b_lin, W_e1, b_e1, W_e12, b_e12, W_geom, b_geom, W_h, b_h, gn_w, gn_b, gn_ms, W_out1, b_out1, W_out2, b_out2, z, edge_index, batch):
    raise NotImplementedError("write your pallas kernel here")


# speedup vs baseline: 2.7873x; 2.7873x over previous
import functools

import numpy as np
import jax
import jax.numpy as jnp
from jax import lax
from jax.experimental import pallas as pl
from jax.experimental.pallas import tpu as pltpu
from jax.experimental.pallas import tpu_sc as plsc

N = 100000
E = 1600000
NGRAPH = 64
HID = 32
NGAUSS = 8
CUTOFF = 5.0

EG = 128
NG = E // EG
BN = 2000
NB = N // BN
BE = 4000
NEB = E // BE

HALF = N // 2
TPS = 16
ROWS_PER_TILE = 3136
ZROWS = 392

_OFFS = np.linspace(0.0, CUTOFF, NGAUSS).astype(np.float32)
_COEFF = np.float32(-0.5 / (_OFFS[1] - _OFFS[0]) ** 2)

_f32 = jnp.float32


def _silu(x):
    return x * (1.0 / (1.0 + jnp.exp(-x)))


_SC_MESH = plsc.VectorSubcoreMesh(
    core_axis_name="c", subcore_axis_name="s", num_cores=2, num_subcores=16)


@functools.partial(
    pl.kernel,
    out_type=(jax.ShapeDtypeStruct((NG, EG, 4), _f32),
              jax.ShapeDtypeStruct((NG, EG, 4), _f32)),
    mesh=_SC_MESH,
    scratch_types=[
        pltpu.VMEM((EG,), jnp.int32),
        pltpu.VMEM((EG,), jnp.int32),
        pltpu.VMEM((EG, 4), _f32),
        pltpu.VMEM((EG, 4), _f32),
        pltpu.SemaphoreType.DMA,
        pltpu.SemaphoreType.DMA,
    ],
    compiler_params=pltpu.CompilerParams(use_tc_tiling_on_sc=False),
)
def _posgather(pos4, src2d, dst2d, outa, outb, sbuf, dbuf, pa, pb, sema, semb):
    c = lax.axis_index("c")
    s = lax.axis_index("s")
    w = s * 2 + c
    rem = NG % 32
    count = jnp.where(w < rem, NG // 32 + 1, NG // 32)
    start = w * (NG // 32) + jnp.minimum(w, rem)

    def body(g, carry):
        gi = start + g
        pltpu.sync_copy(src2d.at[gi], sbuf)
        pltpu.sync_copy(dst2d.at[gi], dbuf)
        da = pltpu.async_copy(pos4.at[sbuf], pa, sema)
        db = pltpu.async_copy(pos4.at[dbuf], pb, semb)
        da.wait()
        db.wait()
        pltpu.sync_copy(pa, outa.at[gi])
        pltpu.sync_copy(pb, outb.at[gi])
        return carry

    lax.fori_loop(0, count, body, 0)


@functools.partial(
    pl.kernel,
    out_type=jax.ShapeDtypeStruct((N, HID), _f32),
    mesh=_SC_MESH,
    scratch_types=[
        pltpu.VMEM((EG,), jnp.int32),
        pltpu.VMEM((EG,), jnp.int32),
        pltpu.VMEM((EG,), jnp.int32),
        pltpu.VMEM((EG, HID), _f32),
        pltpu.VMEM((EG, HID), _f32),
        pltpu.VMEM((ZROWS, HID), _f32),
        pltpu.VMEM_SHARED((TPS * ROWS_PER_TILE, HID), _f32),
        pltpu.SemaphoreType.DMA,
    ],
    compiler_params=pltpu.CompilerParams(use_tc_tiling_on_sc=False),
)
def _message(we3d, src2d, dst2d, h_hbm, agg,
             sbuf, dbuf, idxbuf, webuf, hbuf, zbuf, acc, sem):
    c = lax.axis_index("c")
    s = lax.axis_index("s")
    base = c * HALF
    zeros16 = jnp.zeros((16,), _f32)

    def zrow(r, carry):
        zbuf[r, pl.ds(0, 16)] = zeros16
        zbuf[r, pl.ds(16, 16)] = zeros16
        return carry

    lax.fori_loop(0, ZROWS, zrow, 0)

    def zacc(k, carry):
        pltpu.sync_copy(zbuf, acc.at[pl.ds(s * ROWS_PER_TILE + k * ZROWS, ZROWS), :])
        return carry

    lax.fori_loop(0, ROWS_PER_TILE // ZROWS, zacc, 0)
    plsc.subcore_barrier()

    rem = NG % TPS
    count = jnp.where(s < rem, NG // TPS + 1, NG // TPS)
    start = s * (NG // TPS) + jnp.minimum(s, rem)

    def body(g, carry):
        gi = start + g
        pltpu.sync_copy(src2d.at[gi], sbuf)
        pltpu.sync_copy(dst2d.at[gi], dbuf)
        pltpu.sync_copy(we3d.at[gi], webuf)
        pltpu.async_copy(h_hbm.at[sbuf], hbuf, sem).wait()

        def idxloop(l, carry2):
            d = dbuf[pl.ds(l * 16, 16)]
            loc = d - base
            ok = (loc >= 0) & (loc < HALF)
            idxbuf[pl.ds(l * 16, 16)] = jnp.where(ok, loc, HALF)
            return carry2

        lax.fori_loop(0, EG // 16, idxloop, 0)

        def mrow(r, carry2):
            hbuf[r, pl.ds(0, 16)] = hbuf[r, pl.ds(0, 16)] * webuf[r, pl.ds(0, 16)]
            hbuf[r, pl.ds(16, 16)] = hbuf[r, pl.ds(16, 16)] * webuf[r, pl.ds(16, 16)]
            return carry2

        lax.fori_loop(0, EG, mrow, 0)
        pltpu.sync_copy(hbuf, acc.at[idxbuf], add=True)
        return carry

    lax.fori_loop(0, count, body, 0)
    plsc.subcore_barrier()

    last = HALF - (TPS - 1) * ROWS_PER_TILE

    @pl.when(s < TPS - 1)
    def _copy_full():
        pltpu.sync_copy(
            acc.at[pl.ds(s * ROWS_PER_TILE, ROWS_PER_TILE), :],
            agg.at[pl.ds(base + s * ROWS_PER_TILE, ROWS_PER_TILE), :])

    @pl.when(s == TPS - 1)
    def _copy_last():
        pltpu.sync_copy(
            acc.at[pl.ds((TPS - 1) * ROWS_PER_TILE, last), :],
            agg.at[pl.ds(base + (TPS - 1) * ROWS_PER_TILE, last), :])


def _node_init_body(z3_ref, b3_ref, t2_ref, wo1_ref, bo1_ref, wo2_ref,
                    auxc_ref, h_ref, acc_ref):
    zz = z3_ref[0]
    bb = b3_ref[0]
    ohz = (lax.broadcasted_iota(jnp.int32, (128, BN), 0) == zz).astype(_f32)
    h0 = lax.dot_general(ohz, t2_ref[...], (((0,), (0,)), ((), ())),
                         preferred_element_type=_f32)
    h = _silu(h0)
    h_ref[...] = h
    o = _silu(jnp.dot(h, wo1_ref[...], preferred_element_type=_f32)
              + bo1_ref[...])
    aux = jnp.dot(o, wo2_ref[...], preferred_element_type=_f32) + auxc_ref[...]
    ohb = (lax.broadcasted_iota(jnp.int32, (NGRAPH, BN), 0) == bb).astype(_f32)

    @pl.when(pl.program_id(0) == 0)
    def _():
        acc_ref[...] = jnp.zeros_like(acc_ref)

    acc_ref[...] = acc_ref[...] + jnp.dot(ohb, aux, preferred_element_type=_f32)


_node_init = pl.pallas_call(
    _node_init_body,
    grid=(NB,),
    in_specs=[
        pl.BlockSpec((1, 1, BN), lambda i: (i, 0, 0)),
        pl.BlockSpec((1, 1, BN), lambda i: (i, 0, 0)),
        pl.BlockSpec((128, HID), lambda i: (0, 0)),
        pl.BlockSpec((HID, 16), lambda i: (0, 0)),
        pl.BlockSpec((1, 16), lambda i: (0, 0)),
        pl.BlockSpec((16, 8), lambda i: (0, 0)),
        pl.BlockSpec((1, 8), lambda i: (0, 0)),
    ],
    out_specs=[
        pl.BlockSpec((BN, HID), lambda i: (i, 0)),
        pl.BlockSpec((NGRAPH, 8), lambda i: (0, 0)),
    ],
    out_shape=[
        jax.ShapeDtypeStruct((N, HID), _f32),
        jax.ShapeDtypeStruct((NGRAPH, 8), _f32),
    ],
)


def _edge_embed_body(pa_ref, pb_ref, we1_ref, be1_ref, we12_ref, be12_ref,
                     wg_ref, bg_ref, o0_ref, o1_ref):
    rel = pa_ref[...] - pb_ref[...]
    d2 = jnp.sum(rel * rel, axis=1, keepdims=True)
    dist = jnp.sqrt(d2 + 1e-12)
    step = np.float32(_OFFS[1] - _OFFS[0])
    offs = lax.broadcasted_iota(jnp.int32, (1, NGAUSS), 1).astype(_f32) * step
    ga = jnp.exp(_COEFF * (dist - offs) ** 2)
    e1 = _silu(jnp.dot(rel, we1_ref[...], preferred_element_type=_f32)
               + be1_ref[...])
    e2 = _silu(jnp.dot(ga, we12_ref[...], preferred_element_type=_f32)
               + be12_ref[...])
    e = jnp.concatenate([e1, e2], axis=1)
    o0_ref[...] = _silu(jnp.dot(e, wg_ref[0], preferred_element_type=_f32)
                        + bg_ref[0:1, :])
    o1_ref[...] = _silu(jnp.dot(e, wg_ref[1], preferred_element_type=_f32)
                        + bg_ref[1:2, :])


_edge_embed = pl.pallas_call(
    _edge_embed_body,
    grid=(NEB,),
    in_specs=[
        pl.BlockSpec((BE, 4), lambda i: (i, 0)),
        pl.BlockSpec((BE, 4), lambda i: (i, 0)),
        pl.BlockSpec((4, 16), lambda i: (0, 0)),
        pl.BlockSpec((1, 16), lambda i: (0, 0)),
        pl.BlockSpec((NGAUSS, 16), lambda i: (0, 0)),
        pl.BlockSpec((1, 16), lambda i: (0, 0)),
        pl.BlockSpec((2, HID, HID), lambda i: (0, 0, 0)),
        pl.BlockSpec((2, HID), lambda i: (0, 0)),
    ],
    out_specs=[
        pl.BlockSpec((BE, HID), lambda i: (i, 0)),
        pl.BlockSpec((BE, HID), lambda i: (i, 0)),
    ],
    out_shape=[
        jax.ShapeDtypeStruct((E, HID), _f32),
        jax.ShapeDtypeStruct((E, HID), _f32),
    ],
)


def _stats_body(agg_ref, b3_ref, s_ref):
    a = agg_ref[...]
    bb = b3_ref[0]
    ohb = (lax.broadcasted_iota(jnp.int32, (NGRAPH, BN), 0) == bb).astype(_f32)

    @pl.when(pl.program_id(0) == 0)
    def _():
        s_ref[...] = jnp.zeros_like(s_ref)

    s_ref[0:NGRAPH, :] = s_ref[0:NGRAPH, :] + jnp.dot(
        ohb, a, preferred_element_type=_f32)
    s_ref[NGRAPH:, :] = s_ref[NGRAPH:, :] + jnp.dot(
        ohb, a * a, preferred_element_type=_f32)


_stats = pl.pallas_call(
    _stats_body,
    grid=(NB,),
    in_specs=[
        pl.BlockSpec((BN, HID), lambda i: (i, 0)),
        pl.BlockSpec((1, 1, BN), lambda i: (i, 0, 0)),
    ],
    out_specs=pl.BlockSpec((2 * NGRAPH, HID), lambda i: (0, 0)),
    out_shape=jax.ShapeDtypeStruct((2 * NGRAPH, HID), _f32),
)


def _apply_body(agg_ref, b3_ref, ag_ref, bg_ref, wh_ref, bh_ref, wo1_ref,
                bo1_ref, wo2_ref, auxc_ref, h_ref, acc_ref):
    a = agg_ref[...]
    bb = b3_ref[0]
    ohb = (lax.broadcasted_iota(jnp.int32, (NGRAPH, BN), 0) == bb).astype(_f32)
    ga = lax.dot_general(ohb, ag_ref[...], (((0,), (0,)), ((), ())),
                         preferred_element_type=_f32)
    gb = lax.dot_general(ohb, bg_ref[...], (((0,), (0,)), ((), ())),
                         preferred_element_type=_f32)
    h1 = _silu(ga * a + gb)
    h2 = _silu(jnp.dot(h1, wh_ref[...], preferred_element_type=_f32)
               + bh_ref[...])
    h_ref[...] = h2
    o = _silu(jnp.dot(h2, wo1_ref[...], preferred_element_type=_f32)
              + bo1_ref[...])
    aux = jnp.dot(o, wo2_ref[...], preferred_element_type=_f32) + auxc_ref[...]

    @pl.when(pl.program_id(0) == 0)
    def _():
        acc_ref[...] = jnp.zeros_like(acc_ref)

    acc_ref[...] = acc_ref[...] + jnp.dot(ohb, aux, preferred_element_type=_f32)


_apply = pl.pallas_call(
    _apply_body,
    grid=(NB,),
    in_specs=[
        pl.BlockSpec((BN, HID), lambda i: (i, 0)),
        pl.BlockSpec((1, 1, BN), lambda i: (i, 0, 0)),
        pl.BlockSpec((NGRAPH, HID), lambda i: (0, 0)),
        pl.BlockSpec((NGRAPH, HID), lambda i: (0, 0)),
        pl.BlockSpec((HID, HID), lambda i: (0, 0)),
        pl.BlockSpec((1, HID), lambda i: (0, 0)),
        pl.BlockSpec((HID, 16), lambda i: (0, 0)),
        pl.BlockSpec((1, 16), lambda i: (0, 0)),
        pl.BlockSpec((16, 8), lambda i: (0, 0)),
        pl.BlockSpec((1, 8), lambda i: (0, 0)),
    ],
    out_specs=[
        pl.BlockSpec((BN, HID), lambda i: (i, 0)),
        pl.BlockSpec((NGRAPH, 8), lambda i: (0, 0)),
    ],
    out_shape=[
        jax.ShapeDtypeStruct((N, HID), _f32),
        jax.ShapeDtypeStruct((NGRAPH, 8), _f32),
    ],
)


def kernel(pos, emb_main, emb_tag, emb_period, emb_group, W_lin, b_lin,
           W_e1, b_e1, W_e12, b_e12, W_geom, b_geom, W_h, b_h, gn_w, gn_b,
           gn_ms, W_out1, b_out1, W_out2, b_out2, z, edge_index, batch):
    zz = jnp.arange(120, dtype=jnp.int32)
    period = jnp.clip(zz // 18, 0, 9)
    group = zz % 18 + 1
    T = jnp.concatenate([
        emb_main,
        jnp.broadcast_to(emb_tag[0:1], (120, 8)),
        emb_period[period],
        emb_group[group],
    ], axis=1)
    T2 = jnp.dot(T, W_lin, preferred_element_type=_f32) + b_lin
    T2p = jnp.zeros((128, HID), _f32).at[:120].set(T2)

    pos4 = jnp.pad(pos, ((0, 0), (0, 1)))
    src2d = edge_index[0].reshape(NG, EG)
    dst2d = edge_index[1].reshape(NG, EG)
    z3 = z.reshape(NB, 1, BN)
    b3 = batch.reshape(NB, 1, BN)

    We1p = jnp.zeros((4, 16), _f32).at[:3].set(W_e1)
    be1r = b_e1.reshape(1, 16)
    be12r = b_e12.reshape(1, 16)
    bo1 = b_out1.reshape(1, 16)
    Wo2p = jnp.zeros((16, 8), _f32).at[:, 0:1].set(W_out2)
    auxc = jnp.zeros((1, 8), _f32).at[0, 0].set(b_out2[0]).at[0, 1].set(1.0)

    h, accA = _node_init(z3, b3, T2p, W_out1, bo1, Wo2p, auxc)
    pa3, pb3 = _posgather(pos4, src2d, dst2d)
    we0, we1 = _edge_embed(pa3.reshape(E, 4), pb3.reshape(E, 4),
                           We1p, be1r, W_e12, be12r, W_geom, b_geom)

    counts = jnp.maximum(accA[:, 1:2], 1.0)
    energy = accA[:, 0:1]

    for i, we in ((0, we0), (1, we1)):
        agg = _message(we.reshape(NG, EG, HID), src2d, dst2d, h)
        S = _stats(agg, b3)
        s1 = S[:NGRAPH]
        s2 = S[NGRAPH:]
        mean = s1 / counts
        gms = gn_ms[i][None, :]
        var = s2 / counts - mean * mean * gms * (2.0 - gms)
        rstd = lax.rsqrt(var + 1e-5)
        a_g = gn_w[i][None, :] * rstd
        b_g = gn_b[i][None, :] - gn_w[i][None, :] * rstd * mean * gms
        h, accD = _apply(agg, b3, a_g, b_g, W_h[i], b_h[i].reshape(1, HID),
                         W_out1, bo1, Wo2p, auxc)
        energy = energy + accD[:, 0:1]

    return energy


# speedup vs baseline: 4.1339x; 1.4831x over previous
import functools

import numpy as np
import jax
import jax.numpy as jnp
from jax import lax
from jax.experimental import pallas as pl
from jax.experimental.pallas import tpu as pltpu
from jax.experimental.pallas import tpu_sc as plsc

N = 100000
E = 1600000
NGRAPH = 64
HID = 32
NGAUSS = 8
CUTOFF = 5.0

EG = 128
SG = 4
E_PAD = 1605632
NG = E_PAD // EG
NSUP = NG // SG
SUP_MSG = NG // 16
SUP_POS = NSUP // 32
BN = 2000
NB = N // BN
BE = 4096
NEB = E_PAD // BE

HALF = N // 2
TPS = 16
ROWS_PER_TILE = 3136
ZROWS = 98

_OFFS = np.linspace(0.0, CUTOFF, NGAUSS).astype(np.float32)
_COEFF = np.float32(-0.5 / (_OFFS[1] - _OFFS[0]) ** 2)

_f32 = jnp.float32


def _silu(x):
    return x * (1.0 / (1.0 + jnp.exp(-x)))


_SC_MESH = plsc.VectorSubcoreMesh(
    core_axis_name="c", subcore_axis_name="s", num_cores=2, num_subcores=16)


@functools.partial(
    pl.kernel,
    out_type=(jax.ShapeDtypeStruct((NG, EG, 4), _f32),
              jax.ShapeDtypeStruct((NG, EG, 4), _f32)),
    mesh=_SC_MESH,
    scratch_types=[
        pltpu.VMEM((2, SG, EG), jnp.int32),
        pltpu.VMEM((2, SG, EG), jnp.int32),
        pltpu.VMEM((2, SG, EG, 4), _f32),
        pltpu.VMEM((2, SG, EG, 4), _f32),
        pltpu.SemaphoreType.DMA,
        pltpu.SemaphoreType.DMA,
    ],
    compiler_params=pltpu.CompilerParams(use_tc_tiling_on_sc=False),
)
def _posgather(pos4, src2d, dst2d, outa, outb, sbuf, dbuf, pa, pb, isem, gsem):
    c = lax.axis_index("c")
    s = lax.axis_index("s")
    w = s * 2 + c
    start = w * SUP_POS
    last = SUP_POS - 1

    def load_idx(sup, b):
        g0 = (start + sup) * SG
        pltpu.async_copy(src2d.at[pl.ds(g0, SG), :], sbuf.at[b], isem)
        pltpu.async_copy(dst2d.at[pl.ds(g0, SG), :], dbuf.at[b], isem)

    def wait_idx(b):
        pltpu.make_async_copy(src2d.at[pl.ds(0, SG), :], sbuf.at[b], isem).wait()
        pltpu.make_async_copy(dst2d.at[pl.ds(0, SG), :], dbuf.at[b], isem).wait()

    def fire_gathers(b):
        for g in range(SG):
            pltpu.async_copy(pos4.at[sbuf.at[b, g]], pa.at[b, g], gsem)
            pltpu.async_copy(pos4.at[dbuf.at[b, g]], pb.at[b, g], gsem)

    def wait_gathers(b):
        for g in range(SG):
            pltpu.make_async_copy(pos4.at[sbuf.at[b, g]], pa.at[b, g], gsem).wait()
            pltpu.make_async_copy(pos4.at[dbuf.at[b, g]], pb.at[b, g], gsem).wait()

    def step(k, b):
        wait_gathers(b)

        @pl.when(k + 1 <= last)
        def _():
            wait_idx(1 - b)
            fire_gathers(1 - b)

        g0 = (start + k) * SG
        pltpu.sync_copy(pa.at[b], outa.at[pl.ds(g0, SG)])
        pltpu.sync_copy(pb.at[b], outb.at[pl.ds(g0, SG)])

        @pl.when(k + 2 <= last)
        def _():
            load_idx(k + 2, b)

    load_idx(0, 0)
    wait_idx(0)
    fire_gathers(0)
    load_idx(1, 1)

    def pair(p, carry):
        step(2 * p, 0)
        step(2 * p + 1, 1)
        return carry

    lax.fori_loop(0, SUP_POS // 2, pair, 0)


@functools.partial(
    pl.kernel,
    out_type=jax.ShapeDtypeStruct((N, HID), _f32),
    mesh=_SC_MESH,
    scratch_types=[
        pltpu.VMEM((2, 1, EG), jnp.int32),
        pltpu.VMEM((2, 1, EG), jnp.int32),
        pltpu.VMEM((1, EG), jnp.int32),
        pltpu.VMEM((2, EG, HID), _f32),
        pltpu.VMEM((2, EG, HID), _f32),
        pltpu.VMEM((ZROWS, HID), _f32),
        pltpu.VMEM_SHARED((TPS * ROWS_PER_TILE, HID), _f32),
        pltpu.SemaphoreType.DMA,
        pltpu.SemaphoreType.DMA,
    ],
    compiler_params=pltpu.CompilerParams(use_tc_tiling_on_sc=False),
)
def _message(we3d, src2d, dst2d, h_hbm, agg,
             sbuf, dbuf, idxbuf, webuf, hbuf, zbuf, acc, isem, gsem):
    c = lax.axis_index("c")
    s = lax.axis_index("s")
    base = c * HALF
    zeros16 = jnp.zeros((16,), _f32)

    def zrow(r, carry):
        zbuf[r, pl.ds(0, 16)] = zeros16
        zbuf[r, pl.ds(16, 16)] = zeros16
        return carry

    lax.fori_loop(0, ZROWS, zrow, 0)

    def zacc(k, carry):
        pltpu.sync_copy(zbuf, acc.at[pl.ds(s * ROWS_PER_TILE + k * ZROWS, ZROWS), :])
        return carry

    lax.fori_loop(0, ROWS_PER_TILE // ZROWS, zacc, 0)
    plsc.subcore_barrier()

    start = s * SUP_MSG
    last = SUP_MSG - 1

    def load_idx(sup, b):
        gi = start + sup
        pltpu.async_copy(src2d.at[pl.ds(gi, 1), :], sbuf.at[b], isem)
        pltpu.async_copy(dst2d.at[pl.ds(gi, 1), :], dbuf.at[b], isem)
        pltpu.async_copy(we3d.at[gi], webuf.at[b], isem)

    def wait_idx(b):
        pltpu.make_async_copy(src2d.at[pl.ds(0, 1), :], sbuf.at[b], isem).wait()
        pltpu.make_async_copy(dst2d.at[pl.ds(0, 1), :], dbuf.at[b], isem).wait()
        pltpu.make_async_copy(we3d.at[0], webuf.at[b], isem).wait()

    def fire_gathers(b):
        pltpu.async_copy(h_hbm.at[sbuf.at[b, 0]], hbuf.at[b], gsem)

    def wait_gathers(b):
        pltpu.make_async_copy(h_hbm.at[sbuf.at[b, 0]], hbuf.at[b], gsem).wait()

    def compute(b):
        def idxloop(l, carry):
            o = l * 16
            d = dbuf[b, 0, pl.ds(o, 16)]
            loc = d - base
            ok = (loc >= 0) & (loc < HALF)
            idxbuf[0, pl.ds(o, 16)] = jnp.where(ok, loc, HALF)
            return carry

        lax.fori_loop(0, EG // 16, idxloop, 0)

        def mrow(r, carry):
            hbuf[b, r, pl.ds(0, 16)] = (hbuf[b, r, pl.ds(0, 16)]
                                        * webuf[b, r, pl.ds(0, 16)])
            hbuf[b, r, pl.ds(16, 16)] = (hbuf[b, r, pl.ds(16, 16)]
                                         * webuf[b, r, pl.ds(16, 16)])
            return carry

        lax.fori_loop(0, EG, mrow, 0)
        pltpu.sync_copy(hbuf.at[b], acc.at[idxbuf.at[0]], add=True)

    def step(k, b):
        wait_gathers(b)

        @pl.when(k + 1 <= last)
        def _():
            wait_idx(1 - b)
            fire_gathers(1 - b)

        compute(b)

        @pl.when(k + 2 <= last)
        def _():
            load_idx(k + 2, b)

    load_idx(0, 0)
    wait_idx(0)
    fire_gathers(0)
    load_idx(1, 1)

    def pair(p, carry):
        step(2 * p, 0)
        step(2 * p + 1, 1)
        return carry

    lax.fori_loop(0, SUP_MSG // 2, pair, 0)
    plsc.subcore_barrier()

    last = HALF - (TPS - 1) * ROWS_PER_TILE

    @pl.when(s < TPS - 1)
    def _copy_full():
        pltpu.sync_copy(
            acc.at[pl.ds(s * ROWS_PER_TILE, ROWS_PER_TILE), :],
            agg.at[pl.ds(base + s * ROWS_PER_TILE, ROWS_PER_TILE), :])

    @pl.when(s == TPS - 1)
    def _copy_last():
        pltpu.sync_copy(
            acc.at[pl.ds((TPS - 1) * ROWS_PER_TILE, last), :],
            agg.at[pl.ds(base + (TPS - 1) * ROWS_PER_TILE, last), :])


def _node_init_body(z3_ref, b3_ref, t2_ref, wo1_ref, bo1_ref, wo2_ref,
                    auxc_ref, h_ref, acc_ref):
    zz = z3_ref[0]
    bb = b3_ref[0]
    ohz = (lax.broadcasted_iota(jnp.int32, (128, BN), 0) == zz).astype(_f32)
    h0 = lax.dot_general(ohz, t2_ref[...], (((0,), (0,)), ((), ())),
                         preferred_element_type=_f32)
    h = _silu(h0)
    h_ref[...] = h
    o = _silu(jnp.dot(h, wo1_ref[...], preferred_element_type=_f32)
              + bo1_ref[...])
    aux = jnp.dot(o, wo2_ref[...], preferred_element_type=_f32) + auxc_ref[...]
    ohb = (lax.broadcasted_iota(jnp.int32, (NGRAPH, BN), 0) == bb).astype(_f32)

    @pl.when(pl.program_id(0) == 0)
    def _():
        acc_ref[...] = jnp.zeros_like(acc_ref)

    acc_ref[...] = acc_ref[...] + jnp.dot(ohb, aux, preferred_element_type=_f32)


_node_init = pl.pallas_call(
    _node_init_body,
    grid=(NB,),
    in_specs=[
        pl.BlockSpec((1, 1, BN), lambda i: (i, 0, 0)),
        pl.BlockSpec((1, 1, BN), lambda i: (i, 0, 0)),
        pl.BlockSpec((128, HID), lambda i: (0, 0)),
        pl.BlockSpec((HID, 16), lambda i: (0, 0)),
        pl.BlockSpec((1, 16), lambda i: (0, 0)),
        pl.BlockSpec((16, 8), lambda i: (0, 0)),
        pl.BlockSpec((1, 8), lambda i: (0, 0)),
    ],
    out_specs=[
        pl.BlockSpec((BN, HID), lambda i: (i, 0)),
        pl.BlockSpec((NGRAPH, 8), lambda i: (0, 0)),
    ],
    out_shape=[
        jax.ShapeDtypeStruct((N, HID), _f32),
        jax.ShapeDtypeStruct((NGRAPH, 8), _f32),
    ],
)


def _edge_embed_body(pa_ref, pb_ref, we1_ref, be1_ref, we12_ref, be12_ref,
                     wg_ref, bg_ref, o0_ref, o1_ref):
    rel = pa_ref[...] - pb_ref[...]
    d2 = jnp.sum(rel * rel, axis=1, keepdims=True)
    dist = jnp.sqrt(d2 + 1e-12)
    step = np.float32(_OFFS[1] - _OFFS[0])
    offs = lax.broadcasted_iota(jnp.int32, (1, NGAUSS), 1).astype(_f32) * step
    ga = jnp.exp(_COEFF * (dist - offs) ** 2)
    e1 = _silu(jnp.dot(rel, we1_ref[...], preferred_element_type=_f32)
               + be1_ref[...])
    e2 = _silu(jnp.dot(ga, we12_ref[...], preferred_element_type=_f32)
               + be12_ref[...])
    e = jnp.concatenate([e1, e2], axis=1)
    o0_ref[...] = _silu(jnp.dot(e, wg_ref[0], preferred_element_type=_f32)
                        + bg_ref[0:1, :])
    o1_ref[...] = _silu(jnp.dot(e, wg_ref[1], preferred_element_type=_f32)
                        + bg_ref[1:2, :])


_edge_embed = pl.pallas_call(
    _edge_embed_body,
    grid=(NEB,),
    in_specs=[
        pl.BlockSpec((BE, 4), lambda i: (i, 0)),
        pl.BlockSpec((BE, 4), lambda i: (i, 0)),
        pl.BlockSpec((4, 16), lambda i: (0, 0)),
        pl.BlockSpec((1, 16), lambda i: (0, 0)),
        pl.BlockSpec((NGAUSS, 16), lambda i: (0, 0)),
        pl.BlockSpec((1, 16), lambda i: (0, 0)),
        pl.BlockSpec((2, HID, HID), lambda i: (0, 0, 0)),
        pl.BlockSpec((2, HID), lambda i: (0, 0)),
    ],
    out_specs=[
        pl.BlockSpec((BE, HID), lambda i: (i, 0)),
        pl.BlockSpec((BE, HID), lambda i: (i, 0)),
    ],
    out_shape=[
        jax.ShapeDtypeStruct((E_PAD, HID), _f32),
        jax.ShapeDtypeStruct((E_PAD, HID), _f32),
    ],
)


def _stats_body(agg_ref, b3_ref, s_ref):
    a = agg_ref[...]
    bb = b3_ref[0]
    ohb = (lax.broadcasted_iota(jnp.int32, (NGRAPH, BN), 0) == bb).astype(_f32)

    @pl.when(pl.program_id(0) == 0)
    def _():
        s_ref[...] = jnp.zeros_like(s_ref)

    s_ref[0:NGRAPH, :] = s_ref[0:NGRAPH, :] + jnp.dot(
        ohb, a, preferred_element_type=_f32)
    s_ref[NGRAPH:, :] = s_ref[NGRAPH:, :] + jnp.dot(
        ohb, a * a, preferred_element_type=_f32)


_stats = pl.pallas_call(
    _stats_body,
    grid=(NB,),
    in_specs=[
        pl.BlockSpec((BN, HID), lambda i: (i, 0)),
        pl.BlockSpec((1, 1, BN), lambda i: (i, 0, 0)),
    ],
    out_specs=pl.BlockSpec((2 * NGRAPH, HID), lambda i: (0, 0)),
    out_shape=jax.ShapeDtypeStruct((2 * NGRAPH, HID), _f32),
)


def _apply_body(agg_ref, b3_ref, ag_ref, bg_ref, wh_ref, bh_ref, wo1_ref,
                bo1_ref, wo2_ref, auxc_ref, h_ref, acc_ref):
    a = agg_ref[...]
    bb = b3_ref[0]
    ohb = (lax.broadcasted_iota(jnp.int32, (NGRAPH, BN), 0) == bb).astype(_f32)
    ga = lax.dot_general(ohb, ag_ref[...], (((0,), (0,)), ((), ())),
                         preferred_element_type=_f32)
    gb = lax.dot_general(ohb, bg_ref[...], (((0,), (0,)), ((), ())),
                         preferred_element_type=_f32)
    h1 = _silu(ga * a + gb)
    h2 = _silu(jnp.dot(h1, wh_ref[...], preferred_element_type=_f32)
               + bh_ref[...])
    h_ref[...] = h2
    o = _silu(jnp.dot(h2, wo1_ref[...], preferred_element_type=_f32)
              + bo1_ref[...])
    aux = jnp.dot(o, wo2_ref[...], preferred_element_type=_f32) + auxc_ref[...]

    @pl.when(pl.program_id(0) == 0)
    def _():
        acc_ref[...] = jnp.zeros_like(acc_ref)

    acc_ref[...] = acc_ref[...] + jnp.dot(ohb, aux, preferred_element_type=_f32)


_apply = pl.pallas_call(
    _apply_body,
    grid=(NB,),
    in_specs=[
        pl.BlockSpec((BN, HID), lambda i: (i, 0)),
        pl.BlockSpec((1, 1, BN), lambda i: (i, 0, 0)),
        pl.BlockSpec((NGRAPH, HID), lambda i: (0, 0)),
        pl.BlockSpec((NGRAPH, HID), lambda i: (0, 0)),
        pl.BlockSpec((HID, HID), lambda i: (0, 0)),
        pl.BlockSpec((1, HID), lambda i: (0, 0)),
        pl.BlockSpec((HID, 16), lambda i: (0, 0)),
        pl.BlockSpec((1, 16), lambda i: (0, 0)),
        pl.BlockSpec((16, 8), lambda i: (0, 0)),
        pl.BlockSpec((1, 8), lambda i: (0, 0)),
    ],
    out_specs=[
        pl.BlockSpec((BN, HID), lambda i: (i, 0)),
        pl.BlockSpec((NGRAPH, 8), lambda i: (0, 0)),
    ],
    out_shape=[
        jax.ShapeDtypeStruct((N, HID), _f32),
        jax.ShapeDtypeStruct((NGRAPH, 8), _f32),
    ],
)


def kernel(pos, emb_main, emb_tag, emb_period, emb_group, W_lin, b_lin,
           W_e1, b_e1, W_e12, b_e12, W_geom, b_geom, W_h, b_h, gn_w, gn_b,
           gn_ms, W_out1, b_out1, W_out2, b_out2, z, edge_index, batch):
    zz = jnp.arange(120, dtype=jnp.int32)
    period = jnp.clip(zz // 18, 0, 9)
    group = zz % 18 + 1
    T = jnp.concatenate([
        emb_main,
        jnp.broadcast_to(emb_tag[0:1], (120, 8)),
        emb_period[period],
        emb_group[group],
    ], axis=1)
    T2 = jnp.dot(T, W_lin, preferred_element_type=_f32) + b_lin
    T2p = jnp.zeros((128, HID), _f32).at[:120].set(T2)

    pos4 = jnp.pad(pos, ((0, 0), (0, 1)))
    src2d = jnp.pad(edge_index[0], (0, E_PAD - E)).reshape(NG, EG)
    dst2d = jnp.pad(edge_index[1], (0, E_PAD - E),
                    constant_values=N).reshape(NG, EG)
    z3 = z.reshape(NB, 1, BN)
    b3 = batch.reshape(NB, 1, BN)

    We1p = jnp.zeros((4, 16), _f32).at[:3].set(W_e1)
    be1r = b_e1.reshape(1, 16)
    be12r = b_e12.reshape(1, 16)
    bo1 = b_out1.reshape(1, 16)
    Wo2p = jnp.zeros((16, 8), _f32).at[:, 0:1].set(W_out2)
    auxc = jnp.zeros((1, 8), _f32).at[0, 0].set(b_out2[0]).at[0, 1].set(1.0)

    h, accA = _node_init(z3, b3, T2p, W_out1, bo1, Wo2p, auxc)
    pa3, pb3 = _posgather(pos4, src2d, dst2d)
    we0, we1 = _edge_embed(pa3.reshape(E_PAD, 4), pb3.reshape(E_PAD, 4),
                           We1p, be1r, W_e12, be12r, W_geom, b_geom)

    counts = jnp.maximum(accA[:, 1:2], 1.0)
    energy = accA[:, 0:1]

    for i, we in ((0, we0), (1, we1)):
        agg = _message(we.reshape(NG, EG, HID), src2d, dst2d, h)
        S = _stats(agg, b3)
        s1 = S[:NGRAPH]
        s2 = S[NGRAPH:]
        mean = s1 / counts
        gms = gn_ms[i][None, :]
        var = s2 / counts - mean * mean * gms * (2.0 - gms)
        rstd = lax.rsqrt(var + 1e-5)
        a_g = gn_w[i][None, :] * rstd
        b_g = gn_b[i][None, :] - gn_w[i][None, :] * rstd * mean * gms
        h, accD = _apply(agg, b3, a_g, b_g, W_h[i], b_h[i].reshape(1, HID),
                         W_out1, bo1, Wo2p, auxc)
        energy = energy + accD[:, 0:1]

    return energy
